# baseline (speedup 1.0000x reference)
import jax
import jax.numpy as jnp
from jax import lax
from jax.experimental import pallas as pl
from jax.experimental.pallas import tpu as pltpu
from jax.experimental.pallas import tpu_sc as plsc

N = 10000
E = 160000
D = 256
H = 128
NC = 2
NS = 16
L = 16

NPAD = 10240
SUB = NPAD // NS
CH = 128
NCH = 80
EPAD = NS * NCH * CH
ET1 = EPAD // NS
WCH = 128
RB = 1000


DROW = NPAD // H


def _deg_dis_body(dst_hbm, dis_out, dst_v, acc2, idx_v, red_v, deg_s):
    c = lax.axis_index("c")
    s = lax.axis_index("s")

    @pl.when(c == 0)
    def _():
        pltpu.sync_copy(dst_hbm.at[pl.ds(s * ET1, ET1)], dst_v)
        zero = jnp.zeros((L,), jnp.float32)

        @pl.loop(0, DROW, unroll=4)
        def _(i):
            for m in range(H // L):
                acc2[i, pl.ds(m * L, L)] = zero

        @pl.when(s == 0)
        def _():
            pltpu.sync_copy(acc2, deg_s)

        base16 = lax.iota(jnp.int32, L)

        @pl.loop(0, DROW // L)
        def _(i):
            idx_v[0, pl.ds(i * L, L)] = base16 + i * L

        plsc.subcore_barrier()

        ones = jnp.ones((L,), jnp.float32)

        @pl.loop(0, ET1 // L, unroll=4)
        def _(i):
            v = dst_v[pl.ds(i * L, L)]
            plsc.addupdate_scatter(acc2, [v >> 7, v & 127], ones)

        pltpu.sync_copy(acc2, deg_s.at[idx_v.at[0]], add=True)
        plsc.subcore_barrier()
        pltpu.sync_copy(deg_s, acc2)

        @pl.loop(0, SUB // H)
        def _(r):
            row = s * (SUB // H) + r
            for m in range(H // L):
                d = acc2[row, pl.ds(m * L, L)]
                half = d * 0.5
                bi = plsc.bitcast(d, jnp.int32)
                bi = jnp.int32(0x5F3759DF) - (bi >> 1)
                y = plsc.bitcast(bi, jnp.float32)
                y = y * (1.5 - half * y * y)
                y = y * (1.5 - half * y * y)
                y = y * (1.5 - half * y * y)
                y = jnp.where(d > 0.0, y, 0.0)
                red_v[pl.ds(r * H + m * L, L)] = y

        pltpu.sync_copy(red_v, dis_out.at[pl.ds(s * SUB, SUB)])


_deg_dis = pl.kernel(
    _deg_dis_body,
    out_type=jax.ShapeDtypeStruct((NPAD,), jnp.float32),
    compiler_params=pltpu.CompilerParams(needs_layout_passes=False),
    mesh=plsc.VectorSubcoreMesh(
        core_axis_name="c", subcore_axis_name="s",
        num_cores=NC, num_subcores=NS,
    ),
    scratch_types=[
        pltpu.VMEM((ET1,), jnp.int32),
        pltpu.VMEM((DROW, H), jnp.float32),
        pltpu.VMEM((1, DROW), jnp.int32),
        pltpu.VMEM((SUB,), jnp.float32),
        pltpu.VMEM_SHARED((DROW, H), jnp.float32),
    ],
)


def _mm_body(x_ref, w_ref, dis_ref, h0_ref, h1_ref):
    h = jnp.dot(x_ref[...], w_ref[...], preferred_element_type=jnp.float32)
    h = h * dis_ref[...]
    h0_ref[...] = h[:, :H]
    h1_ref[...] = h[:, H:]


_mm = pl.pallas_call(
    _mm_body,
    grid=(N // RB,),
    in_specs=[
        pl.BlockSpec((RB, D), lambda i: (i, 0)),
        pl.BlockSpec((D, D), lambda i: (0, 0)),
        pl.BlockSpec((RB, 1), lambda i: (i, 0)),
    ],
    out_specs=[
        pl.BlockSpec((RB, H), lambda i: (i, 0)),
        pl.BlockSpec((RB, H), lambda i: (i, 0)),
    ],
    out_shape=[
        jax.ShapeDtypeStruct((N, H), jnp.float32),
        jax.ShapeDtypeStruct((N, H), jnp.float32),
    ],
)


GRP = 8
NG = NCH // GRP


def _edge_body(h0, h1, src_hbm, dst_hbm, s_out,
               src_v, dstb0, dstb1, buf0, buf1,
               gsem0, gsem1, isem0, isem1, acc_s):
    c = lax.axis_index("c")
    s = lax.axis_index("s")

    pltpu.sync_copy(src_hbm.at[s], src_v)

    zero = jnp.zeros((L,), jnp.float32)

    @pl.loop(0, WCH)
    def _(i):
        @pl.loop(0, H // L)
        def _(k):
            buf0[i, pl.ds(k * L, L)] = zero

    @pl.loop(0, SUB // WCH)
    def _(r):
        pltpu.sync_copy(buf0, acc_s.at[pl.ds(s * SUB + r * WCH, WCH)])

    plsc.subcore_barrier()

    def run_edges(h_ref):
        pltpu.async_copy(dst_hbm.at[s].at[pl.ds(0, GRP)], dstb0, isem0)
        pltpu.async_copy(dst_hbm.at[s].at[pl.ds(GRP, GRP)], dstb1, isem1)
        pltpu.async_copy(h_ref.at[src_v.at[0]], buf0, gsem0)
        pltpu.async_copy(h_ref.at[src_v.at[1]], buf1, gsem1)

        @pl.loop(0, NG, step=2)
        def _(g):
            for half, dstb, isem in ((0, dstb0, isem0), (1, dstb1, isem1)):
                gg = g + half
                pltpu.make_async_copy(
                    dst_hbm.at[s].at[pl.ds(gg * GRP, GRP)], dstb, isem
                ).wait()
                for r in range(GRP):
                    jj = gg * GRP + r
                    buf, gsem = (buf0, gsem0) if r % 2 == 0 else (buf1, gsem1)
                    bufs = buf
                    pltpu.make_async_copy(
                        h_ref.at[src_v.at[jj]], bufs, gsem
                    ).wait()
                    pltpu.sync_copy(bufs, acc_s.at[dstb.at[r]], add=True)

                    @pl.when(jj + 2 < NCH)
                    def _():
                        pltpu.async_copy(
                            h_ref.at[src_v.at[jj + 2]], bufs, gsem
                        )

                @pl.when(gg + 2 < NG)
                def _():
                    pltpu.async_copy(
                        dst_hbm.at[s].at[pl.ds((gg + 2) * GRP, GRP)],
                        dstb, isem,
                    )

    @pl.when(c == 0)
    def _():
        run_edges(h0)

    @pl.when(c == 1)
    def _():
        run_edges(h1)

    plsc.subcore_barrier()

    base = s * SUB
    pltpu.sync_copy(acc_s.at[pl.ds(base, SUB)],
                    s_out.at[c].at[pl.ds(base, SUB)])


_edge = pl.kernel(
    _edge_body,
    out_type=jax.ShapeDtypeStruct((NC, NPAD, H), jnp.float32),
    compiler_params=pltpu.CompilerParams(needs_layout_passes=False),
    mesh=plsc.VectorSubcoreMesh(
        core_axis_name="c", subcore_axis_name="s",
        num_cores=NC, num_subcores=NS,
    ),
    scratch_types=[
        pltpu.VMEM((NCH, CH), jnp.int32),
        pltpu.VMEM((GRP, CH), jnp.int32),
        pltpu.VMEM((GRP, CH), jnp.int32),
        pltpu.VMEM((WCH, H), jnp.float32),
        pltpu.VMEM((WCH, H), jnp.float32),
        pltpu.SemaphoreType.DMA,
        pltpu.SemaphoreType.DMA,
        pltpu.SemaphoreType.DMA,
        pltpu.SemaphoreType.DMA,
        pltpu.VMEM_SHARED((NPAD, H), jnp.float32),
    ],
)


def _final_body(s0_ref, s1_ref, dis_ref, b_ref, o_ref):
    d = dis_ref[...]
    o = jnp.concatenate([s0_ref[0] * d, s1_ref[0] * d], axis=1) + b_ref[...]
    o_ref[...] = jnp.maximum(o, 0.0)


_final = pl.pallas_call(
    _final_body,
    grid=(N // RB,),
    in_specs=[
        pl.BlockSpec((1, RB, H), lambda i: (0, i, 0)),
        pl.BlockSpec((1, RB, H), lambda i: (1, i, 0)),
        pl.BlockSpec((RB, 1), lambda i: (i, 0)),
        pl.BlockSpec((1, D), lambda i: (0, 0)),
    ],
    out_specs=pl.BlockSpec((RB, D), lambda i: (i, 0)),
    out_shape=jax.ShapeDtypeStruct((N, D), jnp.float32),
)


def kernel(x, edge_index, W, b):
    npad = EPAD - E
    src = jnp.concatenate([edge_index[0], jnp.zeros((npad,), jnp.int32)])
    dst = jnp.concatenate([edge_index[1], jnp.full((npad,), N, jnp.int32)])
    dis_pad = _deg_dis(dst)
    dis = dis_pad[:N].reshape(N, 1)
    h0, h1 = _mm(x, W, dis)
    src3 = src.reshape(NS, NCH, CH)
    dst3 = dst.reshape(NS, NCH, CH)
    s_out = _edge(h0, h1, src3, dst3)
    return _final(s_out, s_out, dis, b.reshape(1, D))


# speedup vs baseline: 1.1860x; 1.1860x over previous
import jax
import jax.numpy as jnp
from jax import lax
from jax.experimental import pallas as pl
from jax.experimental.pallas import tpu as pltpu
from jax.experimental.pallas import tpu_sc as plsc

N = 10000
E = 160000
D = 256
H = 128
NC = 2
NS = 16
L = 16

NPAD = 10240
SUB = NPAD // NS
CH = 128
NCH = 80
EPAD = NS * NCH * CH
ET1 = EPAD // NS
WCH = 128
RB = 1000


DROW = NPAD // H


def _deg_dis_body(dst_hbm, dis_out, dst_v, acc2, idx_v, red_v, deg_s):
    c = lax.axis_index("c")
    s = lax.axis_index("s")

    @pl.when(c == 0)
    def _():
        pltpu.sync_copy(dst_hbm.at[s], dst_v)
        zero = jnp.zeros((L,), jnp.float32)

        @pl.loop(0, DROW, unroll=4)
        def _(i):
            for m in range(H // L):
                acc2[i, pl.ds(m * L, L)] = zero

        @pl.when(s == 0)
        def _():
            pltpu.sync_copy(acc2, deg_s)

        base16 = lax.iota(jnp.int32, L)

        @pl.loop(0, DROW // L)
        def _(i):
            idx_v[0, pl.ds(i * L, L)] = base16 + i * L

        plsc.subcore_barrier()

        ones = jnp.ones((L,), jnp.float32)

        @pl.loop(0, NCH, unroll=2)
        def _(i):
            for m in range(CH // L):
                v = dst_v[i, pl.ds(m * L, L)]
                plsc.addupdate_scatter(acc2, [v >> 7, v & 127], ones)

        pltpu.sync_copy(acc2, deg_s.at[idx_v.at[0]], add=True)
        plsc.subcore_barrier()
        pltpu.sync_copy(deg_s, acc2)

        @pl.loop(0, SUB // H)
        def _(r):
            row = s * (SUB // H) + r
            for m in range(H // L):
                d = acc2[row, pl.ds(m * L, L)]
                half = d * 0.5
                bi = plsc.bitcast(d, jnp.int32)
                bi = jnp.int32(0x5F3759DF) - (bi >> 1)
                y = plsc.bitcast(bi, jnp.float32)
                y = y * (1.5 - half * y * y)
                y = y * (1.5 - half * y * y)
                y = y * (1.5 - half * y * y)
                y = jnp.where(d > 0.0, y, 0.0)
                red_v[pl.ds(r * H + m * L, L)] = y

        pltpu.sync_copy(red_v, dis_out.at[pl.ds(s * SUB, SUB)])


_deg_dis = pl.kernel(
    _deg_dis_body,
    out_type=jax.ShapeDtypeStruct((NPAD,), jnp.float32),
    compiler_params=pltpu.CompilerParams(needs_layout_passes=False),
    mesh=plsc.VectorSubcoreMesh(
        core_axis_name="c", subcore_axis_name="s",
        num_cores=NC, num_subcores=NS,
    ),
    scratch_types=[
        pltpu.VMEM((NCH, CH), jnp.int32),
        pltpu.VMEM((DROW, H), jnp.float32),
        pltpu.VMEM((1, DROW), jnp.int32),
        pltpu.VMEM((SUB,), jnp.float32),
        pltpu.VMEM_SHARED((DROW, H), jnp.float32),
    ],
)


def _mm_body(x_ref, w_ref, dis_ref, h0_ref, h1_ref):
    h = jnp.dot(x_ref[...], w_ref[...], preferred_element_type=jnp.float32)
    h = h * dis_ref[...]
    h0_ref[...] = h[:, :H]
    h1_ref[...] = h[:, H:]


_mm = pl.pallas_call(
    _mm_body,
    grid=(N // RB,),
    in_specs=[
        pl.BlockSpec((RB, D), lambda i: (i, 0)),
        pl.BlockSpec((D, D), lambda i: (0, 0)),
        pl.BlockSpec((RB, 1), lambda i: (i, 0)),
    ],
    out_specs=[
        pl.BlockSpec((RB, H), lambda i: (i, 0)),
        pl.BlockSpec((RB, H), lambda i: (i, 0)),
    ],
    out_shape=[
        jax.ShapeDtypeStruct((N, H), jnp.float32),
        jax.ShapeDtypeStruct((N, H), jnp.float32),
    ],
)


GRP = 8
NG = NCH // GRP


def _edge_body(h0, h1, src_hbm, dst_hbm, s_out,
               src_v, dstb0, dstb1, buf0, buf1,
               gsem0, gsem1, isem0, isem1, acc_s):
    c = lax.axis_index("c")
    s = lax.axis_index("s")

    pltpu.sync_copy(src_hbm.at[s], src_v)

    zero = jnp.zeros((L,), jnp.float32)

    @pl.loop(0, WCH)
    def _(i):
        @pl.loop(0, H // L)
        def _(k):
            buf0[i, pl.ds(k * L, L)] = zero

    @pl.loop(0, SUB // WCH)
    def _(r):
        pltpu.sync_copy(buf0, acc_s.at[pl.ds(s * SUB + r * WCH, WCH)])

    plsc.subcore_barrier()

    def run_edges(h_ref):
        pltpu.async_copy(dst_hbm.at[s].at[pl.ds(0, GRP)], dstb0, isem0)
        pltpu.async_copy(dst_hbm.at[s].at[pl.ds(GRP, GRP)], dstb1, isem1)
        pltpu.async_copy(h_ref.at[src_v.at[0]], buf0, gsem0)
        pltpu.async_copy(h_ref.at[src_v.at[1]], buf1, gsem1)

        @pl.loop(0, NG, step=2)
        def _(g):
            for half, dstb, isem in ((0, dstb0, isem0), (1, dstb1, isem1)):
                gg = g + half
                pltpu.make_async_copy(
                    dst_hbm.at[s].at[pl.ds(gg * GRP, GRP)], dstb, isem
                ).wait()
                for r in range(GRP):
                    jj = gg * GRP + r
                    buf, gsem = (buf0, gsem0) if r % 2 == 0 else (buf1, gsem1)
                    bufs = buf
                    pltpu.make_async_copy(
                        h_ref.at[src_v.at[jj]], bufs, gsem
                    ).wait()
                    pltpu.sync_copy(bufs, acc_s.at[dstb.at[r]], add=True)

                    @pl.when(jj + 2 < NCH)
                    def _():
                        pltpu.async_copy(
                            h_ref.at[src_v.at[jj + 2]], bufs, gsem
                        )

                @pl.when(gg + 2 < NG)
                def _():
                    pltpu.async_copy(
                        dst_hbm.at[s].at[pl.ds((gg + 2) * GRP, GRP)],
                        dstb, isem,
                    )

    @pl.when(c == 0)
    def _():
        run_edges(h0)

    @pl.when(c == 1)
    def _():
        run_edges(h1)

    plsc.subcore_barrier()

    base = s * SUB
    pltpu.sync_copy(acc_s.at[pl.ds(base, SUB)],
                    s_out.at[c].at[pl.ds(base, SUB)])


_edge = pl.kernel(
    _edge_body,
    out_type=jax.ShapeDtypeStruct((NC, NPAD, H), jnp.float32),
    compiler_params=pltpu.CompilerParams(needs_layout_passes=False),
    mesh=plsc.VectorSubcoreMesh(
        core_axis_name="c", subcore_axis_name="s",
        num_cores=NC, num_subcores=NS,
    ),
    scratch_types=[
        pltpu.VMEM((NCH, CH), jnp.int32),
        pltpu.VMEM((GRP, CH), jnp.int32),
        pltpu.VMEM((GRP, CH), jnp.int32),
        pltpu.VMEM((WCH, H), jnp.float32),
        pltpu.VMEM((WCH, H), jnp.float32),
        pltpu.SemaphoreType.DMA,
        pltpu.SemaphoreType.DMA,
        pltpu.SemaphoreType.DMA,
        pltpu.SemaphoreType.DMA,
        pltpu.VMEM_SHARED((NPAD, H), jnp.float32),
    ],
)


def _final_body(s0_ref, s1_ref, dis_ref, b_ref, o_ref):
    d = dis_ref[...]
    o = jnp.concatenate([s0_ref[0] * d, s1_ref[0] * d], axis=1) + b_ref[...]
    o_ref[...] = jnp.maximum(o, 0.0)


_final = pl.pallas_call(
    _final_body,
    grid=(N // RB,),
    in_specs=[
        pl.BlockSpec((1, RB, H), lambda i: (0, i, 0)),
        pl.BlockSpec((1, RB, H), lambda i: (1, i, 0)),
        pl.BlockSpec((RB, 1), lambda i: (i, 0)),
        pl.BlockSpec((1, D), lambda i: (0, 0)),
    ],
    out_specs=pl.BlockSpec((RB, D), lambda i: (i, 0)),
    out_shape=jax.ShapeDtypeStruct((N, D), jnp.float32),
)


def kernel(x, edge_index, W, b):
    padcols = ET1 - E // NS
    src2 = edge_index[0].reshape(NS, E // NS)
    dst2 = edge_index[1].reshape(NS, E // NS)
    psrc = jnp.zeros((NS, padcols), jnp.int32)
    pdst = jnp.broadcast_to(
        N + jnp.arange(padcols, dtype=jnp.int32), (NS, padcols)
    )
    src3 = jnp.concatenate([src2, psrc], axis=1).reshape(NS, NCH, CH)
    dst3 = jnp.concatenate([dst2, pdst], axis=1).reshape(NS, NCH, CH)
    dis_pad = _deg_dis(dst3)
    dis = dis_pad[:N].reshape(N, 1)
    h0, h1 = _mm(x, W, dis)
    s_out = _edge(h0, h1, src3, dst3)
    return _final(s_out, s_out, dis, b.reshape(1, D))


# speedup vs baseline: 2.1519x; 1.8145x over previous
import jax
import jax.numpy as jnp
from jax import lax
from jax.experimental import pallas as pl
from jax.experimental.pallas import tpu as pltpu
from jax.experimental.pallas import tpu_sc as plsc

N = 10000
E = 160000
D = 256
H = 128
NC = 2
NS = 16
L = 16

NPAD = 10240
SUB = NPAD // NS
CH = 128
NCH = 80
EPAD = NS * NCH * CH
ET1 = EPAD // NS
WCH = 128
RB = 1000


DROW = NPAD // H


def _deg_dis_body(dst_hbm, dis_out, dst_v, acc2, idx_v, red_v, deg_s):
    c = lax.axis_index("c")
    s = lax.axis_index("s")

    @pl.when(c == 0)
    def _():
        pltpu.sync_copy(dst_hbm.at[s], dst_v)
        zero = jnp.zeros((L,), jnp.float32)

        @pl.loop(0, DROW, unroll=4)
        def _(i):
            for m in range(H // L):
                acc2[i, pl.ds(m * L, L)] = zero

        @pl.when(s == 0)
        def _():
            pltpu.sync_copy(acc2, deg_s)

        base16 = lax.iota(jnp.int32, L)

        @pl.loop(0, DROW // L)
        def _(i):
            idx_v[0, pl.ds(i * L, L)] = base16 + i * L

        plsc.subcore_barrier()

        ones = jnp.ones((L,), jnp.float32)

        @pl.loop(0, NCH, unroll=2)
        def _(i):
            for m in range(CH // L):
                v = dst_v[i, pl.ds(m * L, L)]
                plsc.addupdate_scatter(acc2, [v >> 7, v & 127], ones)

        pltpu.sync_copy(acc2, deg_s.at[idx_v.at[0]], add=True)
        plsc.subcore_barrier()
        pltpu.sync_copy(deg_s, acc2)

        @pl.loop(0, SUB // H)
        def _(r):
            row = s * (SUB // H) + r
            for m in range(H // L):
                d = acc2[row, pl.ds(m * L, L)]
                half = d * 0.5
                bi = plsc.bitcast(d, jnp.int32)
                bi = jnp.int32(0x5F3759DF) - (bi >> 1)
                y = plsc.bitcast(bi, jnp.float32)
                y = y * (1.5 - half * y * y)
                y = y * (1.5 - half * y * y)
                y = y * (1.5 - half * y * y)
                y = jnp.where(d > 0.0, y, 0.0)
                red_v[pl.ds(r * H + m * L, L)] = y

        pltpu.sync_copy(red_v, dis_out.at[pl.ds(s * SUB, SUB)])


_deg_dis = pl.kernel(
    _deg_dis_body,
    out_type=jax.ShapeDtypeStruct((NPAD,), jnp.float32),
    compiler_params=pltpu.CompilerParams(needs_layout_passes=False),
    mesh=plsc.VectorSubcoreMesh(
        core_axis_name="c", subcore_axis_name="s",
        num_cores=NC, num_subcores=NS,
    ),
    scratch_types=[
        pltpu.VMEM((NCH, CH), jnp.int32),
        pltpu.VMEM((DROW, H), jnp.float32),
        pltpu.VMEM((1, DROW), jnp.int32),
        pltpu.VMEM((SUB,), jnp.float32),
        pltpu.VMEM_SHARED((DROW, H), jnp.float32),
    ],
)


def _mm_body(x_ref, w_ref, dis_ref, h0_ref, h1_ref):
    h = jnp.dot(x_ref[...], w_ref[...], preferred_element_type=jnp.float32)
    h = h * dis_ref[...]
    h0_ref[...] = h[:, :H]
    h1_ref[...] = h[:, H:]


_mm = pl.pallas_call(
    _mm_body,
    grid=(N // RB,),
    in_specs=[
        pl.BlockSpec((RB, D), lambda i: (i, 0)),
        pl.BlockSpec((D, D), lambda i: (0, 0)),
        pl.BlockSpec((RB, 1), lambda i: (i, 0)),
    ],
    out_specs=[
        pl.BlockSpec((RB, H), lambda i: (i, 0)),
        pl.BlockSpec((RB, H), lambda i: (i, 0)),
    ],
    out_shape=[
        jax.ShapeDtypeStruct((N, H), jnp.float32),
        jax.ShapeDtypeStruct((N, H), jnp.float32),
    ],
)


GRP = 8
NG = NCH // GRP


def _edge_body(h0, h1, src_hbm, dst_hbm, s_out,
               src_v, dstb0, dstb1, buf0, buf1,
               gsem0, gsem1, isem0, isem1, acc_s):
    c = lax.axis_index("c")
    s = lax.axis_index("s")

    pltpu.sync_copy(src_hbm.at[s], src_v)

    zero = jnp.zeros((L,), jnp.float32)

    @pl.loop(0, WCH)
    def _(i):
        @pl.loop(0, H // L)
        def _(k):
            buf0[i, pl.ds(k * L, L)] = zero

    @pl.loop(0, SUB // WCH)
    def _(r):
        pltpu.sync_copy(buf0, acc_s.at[pl.ds(s * SUB + r * WCH, WCH)])

    plsc.subcore_barrier()

    def run_edges(h_ref):
        pltpu.async_copy(dst_hbm.at[s].at[pl.ds(0, GRP)], dstb0, isem0)
        pltpu.async_copy(dst_hbm.at[s].at[pl.ds(GRP, GRP)], dstb1, isem1)
        pltpu.async_copy(h_ref.at[src_v.at[0]], buf0, gsem0)
        pltpu.async_copy(h_ref.at[src_v.at[1]], buf1, gsem1)

        @pl.loop(0, NG, step=2)
        def _(g):
            for half, dstb, isem in ((0, dstb0, isem0), (1, dstb1, isem1)):
                gg = g + half
                pltpu.make_async_copy(
                    dst_hbm.at[s].at[pl.ds(gg * GRP, GRP)], dstb, isem
                ).wait()
                for r in range(GRP):
                    jj = gg * GRP + r
                    buf, gsem = (buf0, gsem0) if r % 2 == 0 else (buf1, gsem1)
                    bufs = buf
                    pltpu.make_async_copy(
                        h_ref.at[src_v.at[jj]], bufs, gsem
                    ).wait()
                    pltpu.sync_copy(bufs, acc_s.at[dstb.at[r]], add=True)

                    @pl.when(jj + 2 < NCH)
                    def _():
                        pltpu.async_copy(
                            h_ref.at[src_v.at[jj + 2]], bufs, gsem
                        )

                @pl.when(gg + 2 < NG)
                def _():
                    pltpu.async_copy(
                        dst_hbm.at[s].at[pl.ds((gg + 2) * GRP, GRP)],
                        dstb, isem,
                    )

    @pl.when(c == 0)
    def _():
        run_edges(h0)

    @pl.when(c == 1)
    def _():
        run_edges(h1)

    plsc.subcore_barrier()

    base = s * SUB
    pltpu.sync_copy(acc_s.at[pl.ds(base, SUB)],
                    s_out.at[c].at[pl.ds(base, SUB)])


_edge = pl.kernel(
    _edge_body,
    out_type=jax.ShapeDtypeStruct((NC, NPAD, H), jnp.float32),
    compiler_params=pltpu.CompilerParams(needs_layout_passes=False),
    mesh=plsc.VectorSubcoreMesh(
        core_axis_name="c", subcore_axis_name="s",
        num_cores=NC, num_subcores=NS,
    ),
    scratch_types=[
        pltpu.VMEM((NCH, CH), jnp.int32),
        pltpu.VMEM((GRP, CH), jnp.int32),
        pltpu.VMEM((GRP, CH), jnp.int32),
        pltpu.VMEM((WCH, H), jnp.float32),
        pltpu.VMEM((WCH, H), jnp.float32),
        pltpu.SemaphoreType.DMA,
        pltpu.SemaphoreType.DMA,
        pltpu.SemaphoreType.DMA,
        pltpu.SemaphoreType.DMA,
        pltpu.VMEM_SHARED((NPAD, H), jnp.float32),
    ],
)


def _final_body(s0_ref, s1_ref, dis_ref, b_ref, o_ref):
    d = dis_ref[...]
    o = jnp.concatenate([s0_ref[0] * d, s1_ref[0] * d], axis=1) + b_ref[...]
    o_ref[...] = jnp.maximum(o, 0.0)


_final = pl.pallas_call(
    _final_body,
    grid=(N // RB,),
    in_specs=[
        pl.BlockSpec((1, RB, H), lambda i: (0, i, 0)),
        pl.BlockSpec((1, RB, H), lambda i: (1, i, 0)),
        pl.BlockSpec((RB, 1), lambda i: (i, 0)),
        pl.BlockSpec((1, D), lambda i: (0, 0)),
    ],
    out_specs=pl.BlockSpec((RB, D), lambda i: (i, 0)),
    out_shape=jax.ShapeDtypeStruct((N, D), jnp.float32),
)


def kernel(x, edge_index, W, b):
    padcols = ET1 - E // NS
    src2 = edge_index[0].reshape(NS, E // NS)
    dst2 = edge_index[1].reshape(NS, E // NS)
    psrc = jnp.broadcast_to(
        jnp.arange(padcols, dtype=jnp.int32), (NS, padcols)
    )
    pdst = jnp.broadcast_to(
        N + jnp.arange(padcols, dtype=jnp.int32), (NS, padcols)
    )
    src3 = jnp.concatenate([src2, psrc], axis=1).reshape(NS, NCH, CH)
    dst3 = jnp.concatenate([dst2, pdst], axis=1).reshape(NS, NCH, CH)
    dis_pad = _deg_dis(dst3)
    dis = dis_pad[:N].reshape(N, 1)
    h0, h1 = _mm(x, W, dis)
    s_out = _edge(h0, h1, src3, dst3)
    return _final(s_out, s_out, dis, b.reshape(1, D))


# speedup vs baseline: 2.1938x; 1.0195x over previous
import jax
import jax.numpy as jnp
from jax import lax
from jax.experimental import pallas as pl
from jax.experimental.pallas import tpu as pltpu
from jax.experimental.pallas import tpu_sc as plsc

N = 10000
E = 160000
D = 256
H = 128
NC = 2
NS = 16
L = 16

NPAD = 10240
SUB = NPAD // NS
ET1 = E // NS
CH = 125
NCH = (E // NS) // CH
WCH = 128
RB = 1000


DROW = NPAD // H


def _deg_dis_body(dst_hbm, dis_out, dst_v, acc2, idx_v, red_v, deg_s):
    c = lax.axis_index("c")
    s = lax.axis_index("s")

    @pl.when(c == 0)
    def _():
        pltpu.sync_copy(dst_hbm.at[pl.ds(s * ET1, ET1)], dst_v)
        zero = jnp.zeros((L,), jnp.float32)

        @pl.loop(0, DROW, unroll=4)
        def _(i):
            for m in range(H // L):
                acc2[i, pl.ds(m * L, L)] = zero

        @pl.when(s == 0)
        def _():
            pltpu.sync_copy(acc2, deg_s)

        base16 = lax.iota(jnp.int32, L)

        @pl.loop(0, DROW // L)
        def _(i):
            idx_v[0, pl.ds(i * L, L)] = base16 + i * L

        plsc.subcore_barrier()

        ones = jnp.ones((L,), jnp.float32)

        @pl.loop(0, ET1 // L, unroll=4)
        def _(i):
            v = dst_v[pl.ds(i * L, L)]
            plsc.addupdate_scatter(acc2, [v >> 7, v & 127], ones)

        pltpu.sync_copy(acc2, deg_s.at[idx_v.at[0]], add=True)
        plsc.subcore_barrier()
        pltpu.sync_copy(deg_s, acc2)

        @pl.loop(0, SUB // H)
        def _(r):
            row = s * (SUB // H) + r
            for m in range(H // L):
                d = acc2[row, pl.ds(m * L, L)]
                half = d * 0.5
                bi = plsc.bitcast(d, jnp.int32)
                bi = jnp.int32(0x5F3759DF) - (bi >> 1)
                y = plsc.bitcast(bi, jnp.float32)
                y = y * (1.5 - half * y * y)
                y = y * (1.5 - half * y * y)
                y = y * (1.5 - half * y * y)
                y = jnp.where(d > 0.0, y, 0.0)
                red_v[pl.ds(r * H + m * L, L)] = y

        pltpu.sync_copy(red_v, dis_out.at[pl.ds(s * SUB, SUB)])


_deg_dis = pl.kernel(
    _deg_dis_body,
    out_type=jax.ShapeDtypeStruct((NPAD,), jnp.float32),
    compiler_params=pltpu.CompilerParams(needs_layout_passes=False),
    mesh=plsc.VectorSubcoreMesh(
        core_axis_name="c", subcore_axis_name="s",
        num_cores=NC, num_subcores=NS,
    ),
    scratch_types=[
        pltpu.VMEM((ET1,), jnp.int32),
        pltpu.VMEM((DROW, H), jnp.float32),
        pltpu.VMEM((1, DROW), jnp.int32),
        pltpu.VMEM((SUB,), jnp.float32),
        pltpu.VMEM_SHARED((DROW, H), jnp.float32),
    ],
)


def _mm_body(x_ref, w_ref, dis_ref, h0_ref, h1_ref):
    h = jnp.dot(x_ref[...], w_ref[...], preferred_element_type=jnp.float32)
    h = h * dis_ref[...]
    h0_ref[...] = h[:, :H]
    h1_ref[...] = h[:, H:]


_mm = pl.pallas_call(
    _mm_body,
    grid=(N // RB,),
    in_specs=[
        pl.BlockSpec((RB, D), lambda i: (i, 0)),
        pl.BlockSpec((D, D), lambda i: (0, 0)),
        pl.BlockSpec((RB, 1), lambda i: (i, 0)),
    ],
    out_specs=[
        pl.BlockSpec((RB, H), lambda i: (i, 0)),
        pl.BlockSpec((RB, H), lambda i: (i, 0)),
    ],
    out_shape=[
        jax.ShapeDtypeStruct((N, H), jnp.float32),
        jax.ShapeDtypeStruct((N, H), jnp.float32),
    ],
)


GRP = 8
NG = NCH // GRP


def _edge_body(h0, h1, src_hbm, dst_hbm, s_out,
               src_v, dstb0, dstb1, buf0, buf1,
               gsem0, gsem1, isem0, isem1, acc_s):
    c = lax.axis_index("c")
    s = lax.axis_index("s")

    pltpu.sync_copy(src_hbm.at[s], src_v)

    zero = jnp.zeros((L,), jnp.float32)

    @pl.loop(0, WCH)
    def _(i):
        @pl.loop(0, H // L)
        def _(k):
            buf0[i, pl.ds(k * L, L)] = zero

    @pl.loop(0, SUB // WCH)
    def _(r):
        pltpu.sync_copy(buf0, acc_s.at[pl.ds(s * SUB + r * WCH, WCH)])

    plsc.subcore_barrier()

    def run_edges(h_ref):
        pltpu.async_copy(dst_hbm.at[s].at[pl.ds(0, GRP)], dstb0, isem0)
        pltpu.async_copy(dst_hbm.at[s].at[pl.ds(GRP, GRP)], dstb1, isem1)
        pltpu.async_copy(h_ref.at[src_v.at[0]], buf0.at[pl.ds(0, CH)], gsem0)
        pltpu.async_copy(h_ref.at[src_v.at[1]], buf1.at[pl.ds(0, CH)], gsem1)

        @pl.loop(0, NG, step=2)
        def _(g):
            for half, dstb, isem in ((0, dstb0, isem0), (1, dstb1, isem1)):
                gg = g + half
                pltpu.make_async_copy(
                    dst_hbm.at[s].at[pl.ds(gg * GRP, GRP)], dstb, isem
                ).wait()
                for r in range(GRP):
                    jj = gg * GRP + r
                    buf, gsem = (buf0, gsem0) if r % 2 == 0 else (buf1, gsem1)
                    bufs = buf.at[pl.ds(0, CH)]
                    pltpu.make_async_copy(
                        h_ref.at[src_v.at[jj]], bufs, gsem
                    ).wait()
                    pltpu.sync_copy(bufs, acc_s.at[dstb.at[r]], add=True)

                    @pl.when(jj + 2 < NCH)
                    def _():
                        pltpu.async_copy(
                            h_ref.at[src_v.at[jj + 2]], bufs, gsem
                        )

                @pl.when(gg + 2 < NG)
                def _():
                    pltpu.async_copy(
                        dst_hbm.at[s].at[pl.ds((gg + 2) * GRP, GRP)],
                        dstb, isem,
                    )

    @pl.when(c == 0)
    def _():
        run_edges(h0)

    @pl.when(c == 1)
    def _():
        run_edges(h1)

    plsc.subcore_barrier()

    base = s * SUB
    pltpu.sync_copy(acc_s.at[pl.ds(base, SUB)],
                    s_out.at[c].at[pl.ds(base, SUB)])


_edge = pl.kernel(
    _edge_body,
    out_type=jax.ShapeDtypeStruct((NC, NPAD, H), jnp.float32),
    compiler_params=pltpu.CompilerParams(needs_layout_passes=False),
    mesh=plsc.VectorSubcoreMesh(
        core_axis_name="c", subcore_axis_name="s",
        num_cores=NC, num_subcores=NS,
    ),
    scratch_types=[
        pltpu.VMEM((NCH, CH), jnp.int32),
        pltpu.VMEM((GRP, CH), jnp.int32),
        pltpu.VMEM((GRP, CH), jnp.int32),
        pltpu.VMEM((WCH, H), jnp.float32),
        pltpu.VMEM((WCH, H), jnp.float32),
        pltpu.SemaphoreType.DMA,
        pltpu.SemaphoreType.DMA,
        pltpu.SemaphoreType.DMA,
        pltpu.SemaphoreType.DMA,
        pltpu.VMEM_SHARED((NPAD, H), jnp.float32),
    ],
)


def _final_body(s0_ref, s1_ref, dis_ref, b_ref, o_ref):
    d = dis_ref[...]
    o = jnp.concatenate([s0_ref[0] * d, s1_ref[0] * d], axis=1) + b_ref[...]
    o_ref[...] = jnp.maximum(o, 0.0)


_final = pl.pallas_call(
    _final_body,
    grid=(N // RB,),
    in_specs=[
        pl.BlockSpec((1, RB, H), lambda i: (0, i, 0)),
        pl.BlockSpec((1, RB, H), lambda i: (1, i, 0)),
        pl.BlockSpec((RB, 1), lambda i: (i, 0)),
        pl.BlockSpec((1, D), lambda i: (0, 0)),
    ],
    out_specs=pl.BlockSpec((RB, D), lambda i: (i, 0)),
    out_shape=jax.ShapeDtypeStruct((N, D), jnp.float32),
)


def kernel(x, edge_index, W, b):
    src = edge_index[0]
    dst = edge_index[1]
    dis_pad = _deg_dis(dst)
    dis = dis_pad[:N].reshape(N, 1)
    h0, h1 = _mm(x, W, dis)
    src3 = src.reshape(NS, NCH, CH)
    dst3 = dst.reshape(NS, NCH, CH)
    s_out = _edge(h0, h1, src3, dst3)
    return _final(s_out, s_out, dis, b.reshape(1, D))
